# Initial kernel scaffold; baseline (speedup 1.0000x reference)
import jax
import jax.numpy as jnp
from jax.experimental import pallas as pl


def kernel(x, table):
    raise NotImplementedError("write your pallas kernel here")


# speedup vs baseline: 6.8130x; 6.8130x over previous
import functools
import math

import jax
import jax.numpy as jnp
from jax import lax
from jax.experimental import pallas as pl
from jax.experimental.pallas import tpu as pltpu
from jax.experimental.pallas import tpu_sc as plsc

NC = 2
NS = 16
NW = NC * NS

CHUNK = 512
GATHER = 128
K = CHUNK // GATHER


def _scale_body(t_ref, o_ref, *, scale):
    o_ref[...] = t_ref[...] * scale


def _scale_table(table, scale):
    v, d = table.shape
    rows = 1000
    assert v % rows == 0
    return pl.pallas_call(
        functools.partial(_scale_body, scale=scale),
        grid=(v // rows,),
        in_specs=[pl.BlockSpec((rows, d), lambda i: (i, 0))],
        out_specs=pl.BlockSpec((rows, d), lambda i: (i, 0)),
        out_shape=jax.ShapeDtypeStruct((v, d), table.dtype),
    )(table)


@functools.cache
def _make_gather(b_total, d):
    assert b_total % (NW * CHUNK) == 0
    b_per_w = b_total // NW
    nchunk = b_per_w // CHUNK
    idx_rows_per_w = b_per_w // GATHER

    mesh = plsc.VectorSubcoreMesh(
        core_axis_name="c", subcore_axis_name="s",
        num_cores=NC, num_subcores=NS,
    )

    @functools.partial(
        pl.kernel,
        out_type=jax.ShapeDtypeStruct((b_total, d), jnp.float32),
        mesh=mesh,
        scratch_types=[
            pltpu.VMEM((K, GATHER), jnp.int32),
            pltpu.VMEM((CHUNK, d), jnp.float32),
            pltpu.SemaphoreType.DMA,
        ],
    )
    def gather_kernel(table_hbm, idx_hbm, out_hbm, idx_v, rows_v, sem):
        wid = lax.axis_index("s") * NC + lax.axis_index("c")
        idx_base = wid * idx_rows_per_w
        out_base = wid * b_per_w

        def body(g, carry):
            pltpu.sync_copy(idx_hbm.at[pl.ds(idx_base + g * K, K)], idx_v)
            descs = [
                pltpu.async_copy(
                    table_hbm.at[idx_v.at[j]],
                    rows_v.at[pl.ds(j * GATHER, GATHER)],
                    sem,
                )
                for j in range(K)
            ]
            for desc in descs:
                desc.wait()
            pltpu.sync_copy(rows_v, out_hbm.at[pl.ds(out_base + g * CHUNK, CHUNK)])
            return carry

        lax.fori_loop(0, nchunk, body, 0)

    return gather_kernel


def kernel(x, table):
    d = table.shape[1]
    b_total = x.size
    scale = math.sqrt(d)
    scaled = _scale_table(table, scale)
    idx = x.reshape(b_total // GATHER, GATHER).astype(jnp.int32)
    out = _make_gather(b_total, d)(scaled, idx)
    return out.reshape(x.shape + (d,))


# speedup vs baseline: 7.5190x; 1.1036x over previous
import functools
import math

import jax
import jax.numpy as jnp
from jax import lax
from jax.experimental import pallas as pl
from jax.experimental.pallas import tpu as pltpu
from jax.experimental.pallas import tpu_sc as plsc

NC = 2
NS = 16
NW = NC * NS

CHUNK = 256
GATHER = 128
K = CHUNK // GATHER


def _scale_body(t_ref, o_ref, *, scale):
    o_ref[...] = t_ref[...] * scale


def _scale_table(table, scale):
    v, d = table.shape
    rows = 1000
    assert v % rows == 0
    return pl.pallas_call(
        functools.partial(_scale_body, scale=scale),
        grid=(v // rows,),
        in_specs=[pl.BlockSpec((rows, d), lambda i: (i, 0))],
        out_specs=pl.BlockSpec((rows, d), lambda i: (i, 0)),
        out_shape=jax.ShapeDtypeStruct((v, d), table.dtype),
    )(table)


@functools.cache
def _make_gather(b_total, d):
    assert b_total % (NW * CHUNK) == 0
    b_per_w = b_total // NW
    nchunk = b_per_w // CHUNK
    idx_rows_per_w = b_per_w // GATHER

    mesh = plsc.VectorSubcoreMesh(
        core_axis_name="c", subcore_axis_name="s",
        num_cores=NC, num_subcores=NS,
    )

    assert nchunk % 2 == 0
    npair = nchunk // 2

    @functools.partial(
        pl.kernel,
        out_type=jax.ShapeDtypeStruct((b_total, d), jnp.float32),
        mesh=mesh,
        scratch_types=[
            pltpu.VMEM((K, GATHER), jnp.int32),
            pltpu.VMEM((K, GATHER), jnp.int32),
            pltpu.VMEM((CHUNK, d), jnp.float32),
            pltpu.VMEM((CHUNK, d), jnp.float32),
            pltpu.SemaphoreType.DMA,
            pltpu.SemaphoreType.DMA,
            pltpu.SemaphoreType.DMA,
            pltpu.SemaphoreType.DMA,
        ],
    )
    def gather_kernel(table_hbm, idx_hbm, out_hbm,
                      idx0, idx1, rows0, rows1, gsem0, gsem1, osem0, osem1):
        wid = lax.axis_index("s") * NC + lax.axis_index("c")
        idx_base = wid * idx_rows_per_w
        out_base = wid * b_per_w

        def fire(g, idx_v, rows_v, gsem):
            pltpu.sync_copy(idx_hbm.at[pl.ds(idx_base + g * K, K)], idx_v)
            for j in range(K):
                pltpu.async_copy(
                    table_hbm.at[idx_v.at[j]],
                    rows_v.at[pl.ds(j * GATHER, GATHER)],
                    gsem,
                )

        def drain_gathers(rows_v, gsem):
            for j in range(K):
                pltpu.make_async_copy(
                    table_hbm.at[pl.ds(0, GATHER)],
                    rows_v.at[pl.ds(j * GATHER, GATHER)],
                    gsem,
                ).wait()

        def put(g, rows_v, osem):
            return pltpu.async_copy(
                rows_v, out_hbm.at[pl.ds(out_base + g * CHUNK, CHUNK)], osem)

        def wait_put(rows_v, osem):
            pltpu.make_async_copy(
                rows_v, out_hbm.at[pl.ds(out_base, CHUNK)], osem).wait()

        fire(0, idx0, rows0, gsem0)

        def body(i, carry):
            g0 = 2 * i
            @pl.when(i > 0)
            def _():
                wait_put(rows1, osem1)
            fire(g0 + 1, idx1, rows1, gsem1)
            drain_gathers(rows0, gsem0)
            put(g0, rows0, osem0)
            drain_gathers(rows1, gsem1)
            put(g0 + 1, rows1, osem1)
            wait_put(rows0, osem0)
            @pl.when(i + 1 < npair)
            def _():
                fire(g0 + 2, idx0, rows0, gsem0)
            return carry

        lax.fori_loop(0, npair, body, 0)
        wait_put(rows1, osem1)

    return gather_kernel


def kernel(x, table):
    d = table.shape[1]
    b_total = x.size
    scale = math.sqrt(d)
    scaled = _scale_table(table, scale)
    idx = x.reshape(b_total // GATHER, GATHER).astype(jnp.int32)
    out = _make_gather(b_total, d)(scaled, idx)
    return out.reshape(x.shape + (d,))
